# Initial kernel scaffold; baseline (speedup 1.0000x reference)
import jax
import jax.numpy as jnp
from jax.experimental import pallas as pl


def kernel(idxs, labels):
    raise NotImplementedError("write your pallas kernel here")


# speedup vs baseline: 279.3876x; 279.3876x over previous
import functools

import jax
import jax.numpy as jnp
from jax import lax
from jax.experimental import pallas as pl
from jax.experimental.pallas import tpu as pltpu
from jax.experimental.pallas import tpu_sc as plsc

NC, NS, L = 2, 16, 16
NW = NC * NS

R, C = 16384, 200
N = R * C
PER_W = N // NW
CHUNK = 25_600
NCHUNK = PER_W // CHUNK
TAB = 128

_mesh = plsc.VectorSubcoreMesh(
    core_axis_name="c", subcore_axis_name="s", num_cores=NC, num_subcores=NS
)


@functools.partial(
    pl.kernel,
    out_type=jax.ShapeDtypeStruct((N,), jnp.float32),
    mesh=_mesh,
    scratch_types=[
        pltpu.VMEM((TAB,), jnp.float32),
        pltpu.VMEM((CHUNK,), jnp.int32),
        pltpu.VMEM((CHUNK,), jnp.float32),
    ],
    compiler_params=pltpu.CompilerParams(needs_layout_passes=False),
)
def _lut_sc(idx_hbm, tab_hbm, out_hbm, tab_v, idx_v, out_v):
    wid = lax.axis_index("s") * NC + lax.axis_index("c")
    base = wid * PER_W
    pltpu.sync_copy(tab_hbm, tab_v)
    for c in range(NCHUNK):
        off = base + c * CHUNK
        pltpu.sync_copy(idx_hbm.at[pl.ds(off, CHUNK)], idx_v)

        @plsc.parallel_loop(0, CHUNK, step=L, unroll=8)
        def _(i):
            iv = idx_v[pl.ds(i, L)]
            out_v[pl.ds(i, L)] = plsc.load_gather(tab_v, [iv])

        pltpu.sync_copy(out_v, out_hbm.at[pl.ds(off, CHUNK)])


def kernel(idxs, labels):
    flat = idxs.reshape(N).astype(jnp.int32)
    tab = jnp.zeros((TAB,), jnp.float32).at[: labels.shape[0]].set(labels)
    return _lut_sc(flat, tab).reshape(R, C)
